# baseline (speedup 1.0000x reference)
import jax
import jax.numpy as jnp
from jax import lax
from jax.experimental import pallas as pl
from jax.experimental.pallas import tpu as pltpu
from jax.experimental.pallas import tpu_sc as plsc

T, H, F, E, K = 4096, 1024, 2816, 8, 2
B = 512
P = T * K + E * B
NB = P // B
FT = 256
NF = F // FT
TB = 1024

NC, NS, L = 2, 16, 16
NW = NC * NS
PP = P + 256
CH = PP // NS
TW = T // NS
GR = P // NW
GC = 48
TCW = T // NW
CC = 32

_mesh = plsc.VectorSubcoreMesh(core_axis_name="c", subcore_axis_name="s")


def _router_body(x_ref, gw_ref, ct_ref):
    xb = x_ref[...]
    gw = gw_ref[...]
    lg = jax.lax.dot_general(gw, xb, (((1,), (1,)), ((), ())),
                             preferred_element_type=jnp.float32)
    iota = jax.lax.broadcasted_iota(jnp.int32, (E, TB), 0)
    m1 = jnp.max(lg, axis=0, keepdims=True)
    i1 = jnp.min(jnp.where(lg == m1, iota, E), axis=0, keepdims=True)
    is1 = iota == i1
    masked = jnp.where(is1, -jnp.inf, lg)
    m2 = jnp.max(masked, axis=0, keepdims=True)
    i2 = jnp.min(jnp.where(masked == m2, iota, E), axis=0, keepdims=True)
    r = jnp.exp(m2 - m1)
    wa = 1.0 / (1.0 + r)
    wb = r * wa
    ct = jnp.where(is1, wa, 0.0) + jnp.where(iota == i2, wb, 0.0)
    ct_ref[...] = ct


def _run_router(x, gate_w):
    return pl.pallas_call(
        _router_body,
        grid=(T // TB,),
        in_specs=[
            pl.BlockSpec((TB, H), lambda i: (i, 0)),
            pl.BlockSpec((E, H), lambda i: (0, 0)),
        ],
        out_specs=pl.BlockSpec((E, TB), lambda i: (0, i)),
        out_shape=jax.ShapeDtypeStruct((E, T), jnp.float32),
    )(x, gate_w)


def _routing_body(ct_hbm, tok_hbm, cof_hbm, pos_hbm, enc_hbm,
                  ctbuf, histbuf, allhist, encbuf,
                  posbuf, idxbuf, tokbuf, cofbuf, zib, zfb,
                  hist_sh, tok_sh, cof_sh):
    cid = lax.axis_index("c")
    sid = lax.axis_index("s")

    @pl.when(cid == 0)
    def _():
        t0 = sid * TW
        pltpu.sync_copy(ct_hbm.at[:, pl.ds(t0, TW)], ctbuf)
        iota = lax.iota(jnp.int32, L)
        zi = jnp.zeros((L,), jnp.int32)
        zf = jnp.zeros((L,), jnp.float32)

        histv = zi
        for e in range(E):
            acc = zi
            for g in range(TW // L):
                v = ctbuf[e, pl.ds(g * L, L)]
                acc += jnp.where(v > 0.0, 1, 0)
            histv += jnp.where(iota == e, jnp.sum(acc), 0)
        histbuf[pl.ds(0, L)] = histv
        pltpu.sync_copy(histbuf, hist_sh.at[sid])
        for i in range(CH // L):
            zib[pl.ds(L * i, L)] = zi
            zfb[pl.ds(L * i, L)] = zf
        pltpu.sync_copy(zib, tok_sh.at[pl.ds(CH * sid, CH)])
        pltpu.sync_copy(zfb, cof_sh.at[pl.ds(CH * sid, CH)])
        for i in range((TW * K) // L):
            posbuf[pl.ds(L * i, L)] = zi + (P - 1)
        for i in range(K * TW // (4 * L)):
            for j in range(4):
                idxbuf[i, pl.ds(L * j, L)] = zi + P
        plsc.subcore_barrier()

        pltpu.sync_copy(hist_sh, allhist)
        total = zi
        before = zi
        widv = zi + sid
        for w in range(NS):
            hv = allhist[w]
            total += hv
            before += jnp.where(widv > w, hv, 0)
        pc = ((total + (B - 1)) // B) * B
        offend = plsc.cumsum(pc)
        base = (offend - pc) + before

        @pl.when(sid == 0)
        def _():
            for vb in range(2):
                bstart = (iota + L * vb) * B
                cnt = zi
                for e in range(E):
                    oe = offend.at[jnp.full((L,), e, jnp.int32)].get(
                        mode="promise_in_bounds")
                    cnt += jnp.where(bstart >= oe, 1, 0)
                encv = jnp.where(cnt < E, jnp.minimum(cnt, E - 1), -1)
                encbuf[pl.ds(L * vb, L)] = encv
            pltpu.sync_copy(encbuf, enc_hbm)

        for g in range(TW // L):
            ltok = iota + g * L
            slot = zi
            for e in range(E):
                v = ctbuf[e, pl.ds(g * L, L)]
                sel = v > 0.0
                si = jnp.where(sel, 1, 0)
                basebr = base.at[jnp.full((L,), e, jnp.int32)].get(
                    mode="promise_in_bounds")
                posv = basebr + plsc.cumsum(si) - 1
                sidx = ltok * K + slot
                plsc.store_scatter(posbuf, [sidx], posv, mask=sel)
                plsc.store_scatter(idxbuf, [sidx // (4 * L), sidx % (4 * L)],
                                   posv, mask=sel)
                plsc.store_scatter(tokbuf, [sidx // (4 * L), sidx % (4 * L)],
                                   ltok + t0, mask=sel)
                plsc.store_scatter(cofbuf, [sidx // (4 * L), sidx % (4 * L)],
                                   v, mask=sel)
                slot += si
                base += jnp.where(iota == e, jnp.sum(si), 0)
        for i in range(K * TW // (4 * L)):
            pltpu.sync_copy(tokbuf.at[i], tok_sh.at[idxbuf.at[i]])
            pltpu.sync_copy(cofbuf.at[i], cof_sh.at[idxbuf.at[i]])
        pltpu.sync_copy(posbuf, pos_hbm.at[pl.ds(sid * TW * K, TW * K)])
        plsc.subcore_barrier()

        pltpu.sync_copy(tok_sh.at[pl.ds(CH * sid, CH)], zib)
        pltpu.sync_copy(zib, tok_hbm.at[pl.ds(CH * sid, CH)])
        pltpu.sync_copy(cof_sh.at[pl.ds(CH * sid, CH)], zfb)
        pltpu.sync_copy(zfb, cof_hbm.at[pl.ds(CH * sid, CH)])


def _run_routing(ct):
    kfn = pl.kernel(
        _routing_body,
        out_type=(
            jax.ShapeDtypeStruct((PP,), jnp.int32),
            jax.ShapeDtypeStruct((PP,), jnp.float32),
            jax.ShapeDtypeStruct((T * K,), jnp.int32),
            jax.ShapeDtypeStruct((32,), jnp.int32),
        ),
        mesh=_mesh,
        scratch_types=[
            pltpu.VMEM((E, TW), jnp.float32),
            pltpu.VMEM((L,), jnp.int32),
            pltpu.VMEM((NS, L), jnp.int32),
            pltpu.VMEM((2 * L,), jnp.int32),
            pltpu.VMEM((TW * K,), jnp.int32),
            pltpu.VMEM((K * TW // (4 * L), 4 * L), jnp.int32),
            pltpu.VMEM((K * TW // (4 * L), 4 * L), jnp.int32),
            pltpu.VMEM((K * TW // (4 * L), 4 * L), jnp.float32),
            pltpu.VMEM((CH,), jnp.int32),
            pltpu.VMEM((CH,), jnp.float32),
            pltpu.VMEM_SHARED((NS, L), jnp.int32),
            pltpu.VMEM_SHARED((PP,), jnp.int32),
            pltpu.VMEM_SHARED((PP,), jnp.float32),
        ],
        compiler_params=pltpu.CompilerParams(needs_layout_passes=False),
    )
    return kfn(ct)


def _route_jnp(ct):
    c = ct.T
    mask = c > 0.0
    mi = mask.astype(jnp.int32)
    hist = jnp.sum(mi, axis=0)
    pc = ((hist + B - 1) // B) * B
    offend = jnp.cumsum(pc)
    off = offend - pc
    rank = jnp.cumsum(mi, axis=0) - mi
    posmat = off[None, :] + rank
    scat = jnp.where(mask, posmat, P)
    tvec = jax.lax.broadcasted_iota(jnp.int32, (T, E), 0)
    tok = jnp.zeros((P,), jnp.int32).at[scat.reshape(-1)].set(
        tvec.reshape(-1), mode="drop")
    coeff = jnp.zeros((P,), jnp.float32).at[scat.reshape(-1)].set(
        c.reshape(-1), mode="drop")
    kidx = jnp.cumsum(mi, axis=1) - mi
    prows = jnp.where(mask, tvec * K + kidx, T * K)
    pos = jnp.full((T * K,), P - 1, jnp.int32).at[prows.reshape(-1)].set(
        posmat.reshape(-1).astype(jnp.int32), mode="drop")
    bstart = jnp.arange(NB, dtype=jnp.int32) * B
    be = jnp.sum((bstart[:, None] >= offend[None, :]).astype(jnp.int32), axis=1)
    enc = jnp.where(be < E, jnp.minimum(be, E - 1), -1)
    return tok, coeff, pos, enc


def _gather_body(x_hbm, tok_hbm, xs_hbm, tokw, buf0, buf1, sem0, sem1):
    cid = lax.axis_index("c")
    sid = lax.axis_index("s")
    wid = sid * NC + cid
    r0 = wid * GR
    pltpu.sync_copy(tok_hbm.at[pl.ds(r0, GR)], tokw)
    nchunk = GR // GC
    cps = []
    for j in range(nchunk):
        buf = buf0 if j % 2 == 0 else buf1
        sem = sem0 if j % 2 == 0 else sem1
        cp = pltpu.async_copy(x_hbm.at[tokw.at[pl.ds(j * GC, GC)]], buf, sem)
        cps.append(cp)
        if j >= 1:
            cps[j - 1].wait()
            pb = buf1 if j % 2 == 0 else buf0
            pltpu.sync_copy(pb, xs_hbm.at[pl.ds(r0 + (j - 1) * GC, GC)])
    cps[nchunk - 1].wait()
    lb = buf0 if (nchunk - 1) % 2 == 0 else buf1
    pltpu.sync_copy(lb, xs_hbm.at[pl.ds(r0 + (nchunk - 1) * GC, GC)])


def _run_gather(x, tok):
    kfn = pl.kernel(
        _gather_body,
        out_type=jax.ShapeDtypeStruct((P, H), jnp.float32),
        mesh=_mesh,
        scratch_types=[
            pltpu.VMEM((GR,), jnp.int32),
            pltpu.VMEM((GC, H), jnp.float32),
            pltpu.VMEM((GC, H), jnp.float32),
            pltpu.SemaphoreType.DMA,
            pltpu.SemaphoreType.DMA,
        ],
    )
    return kfn(x, tok)


def _ffn_body(enc_ref, xs_ref, w1_ref, w3_ref, w2_ref, cf_ref, ye_ref):
    f = pl.program_id(1)
    b = pl.program_id(0)
    active = enc_ref[b] >= 0

    @pl.when(f == 0)
    def _():
        ye_ref[...] = jnp.zeros_like(ye_ref)

    @pl.when(active)
    def _():
        xs = xs_ref[...].astype(jnp.bfloat16)
        w1b = w1_ref[0].astype(jnp.bfloat16)
        w3b = w3_ref[0].astype(jnp.bfloat16)
        w2b = w2_ref[0].astype(jnp.bfloat16)
        h1 = jax.lax.dot_general(xs, w1b, (((1,), (1,)), ((), ())),
                                 preferred_element_type=jnp.float32)
        h3 = jax.lax.dot_general(xs, w3b, (((1,), (1,)), ((), ())),
                                 preferred_element_type=jnp.float32)
        g = ((h1 * jax.lax.logistic(h1)) * h3).astype(jnp.bfloat16)
        contrib = jax.lax.dot_general(g, w2b, (((1,), (1,)), ((), ())),
                                      preferred_element_type=jnp.float32)
        ye_ref[...] += contrib

    @pl.when(f == NF - 1)
    def _():
        ye_ref[...] *= cf_ref[...]


def _run_ffn(enc, xs, w1, w3, w2, coeffcol):
    grid_spec = pltpu.PrefetchScalarGridSpec(
        num_scalar_prefetch=1,
        grid=(NB, NF),
        in_specs=[
            pl.BlockSpec((B, H), lambda b, f, s: (b, 0)),
            pl.BlockSpec((1, FT, H),
                         lambda b, f, s: (jnp.maximum(s[b], 0),
                                          jnp.where(s[b] >= 0, f, 0), 0)),
            pl.BlockSpec((1, FT, H),
                         lambda b, f, s: (jnp.maximum(s[b], 0),
                                          jnp.where(s[b] >= 0, f, 0), 0)),
            pl.BlockSpec((1, H, FT),
                         lambda b, f, s: (jnp.maximum(s[b], 0), 0,
                                          jnp.where(s[b] >= 0, f, 0))),
            pl.BlockSpec((B, 1), lambda b, f, s: (b, 0)),
        ],
        out_specs=pl.BlockSpec((B, H), lambda b, f, s: (b, 0)),
    )
    return pl.pallas_call(
        _ffn_body,
        grid_spec=grid_spec,
        out_shape=jax.ShapeDtypeStruct((P, H), jnp.float32),
        compiler_params=pltpu.CompilerParams(
            dimension_semantics=("arbitrary", "arbitrary")),
    )(enc, xs, w1, w3, w2, coeffcol)


def _combine_body(ye_hbm, pos_hbm, out_hbm,
                  posw, ideven, idodd, bufa, bufb, sema, semb):
    cid = lax.axis_index("c")
    sid = lax.axis_index("s")
    wid = sid * NC + cid
    t0 = wid * TCW
    iota = lax.iota(jnp.int32, L)
    pltpu.sync_copy(pos_hbm.at[pl.ds(t0 * K, TCW * K)], posw)
    for g in range(TCW // L):
        base = 2 * L * g
        ideven[pl.ds(L * g, L)] = plsc.load_gather(posw, [base + 2 * iota])
        idodd[pl.ds(L * g, L)] = plsc.load_gather(posw, [base + 2 * iota + 1])
    for j in range(TCW // CC):
        cpa = pltpu.async_copy(
            ye_hbm.at[ideven.at[pl.ds(j * CC, CC)]], bufa, sema)
        cpb = pltpu.async_copy(
            ye_hbm.at[idodd.at[pl.ds(j * CC, CC)]], bufb, semb)
        cpa.wait()
        cpb.wait()

        def _add_row(r, carry):
            for c in range(H // L):
                bufa[r, pl.ds(L * c, L)] = (bufa[r, pl.ds(L * c, L)]
                                            + bufb[r, pl.ds(L * c, L)])
            return carry

        lax.fori_loop(0, CC, _add_row, 0)
        pltpu.sync_copy(bufa, out_hbm.at[pl.ds(t0 + j * CC, CC)])


def _run_combine(ye, pos):
    kfn = pl.kernel(
        _combine_body,
        out_type=jax.ShapeDtypeStruct((T, H), jnp.float32),
        mesh=_mesh,
        scratch_types=[
            pltpu.VMEM((TCW * K,), jnp.int32),
            pltpu.VMEM((TCW,), jnp.int32),
            pltpu.VMEM((TCW,), jnp.int32),
            pltpu.VMEM((CC, H), jnp.float32),
            pltpu.VMEM((CC, H), jnp.float32),
            pltpu.SemaphoreType.DMA,
            pltpu.SemaphoreType.DMA,
        ],
        compiler_params=pltpu.CompilerParams(needs_layout_passes=False),
    )
    return kfn(ye, pos)


@jax.jit
def kernel(x, gate_w, w1, w2, w3):
    ct = _run_router(x, gate_w)
    tok, cof, pos, enc = _route_jnp(ct)
    xs = _run_gather(x, tok)
    ye = _run_ffn(enc, xs, w1, w3, w2, cof[:P, None])
    out = _run_combine(ye, pos)
    return out


# speedup vs baseline: 1.1694x; 1.1694x over previous
import jax
import jax.numpy as jnp
from jax import lax
from jax.experimental import pallas as pl
from jax.experimental.pallas import tpu as pltpu
from jax.experimental.pallas import tpu_sc as plsc

T, H, F, E, K = 4096, 1024, 2816, 8, 2
B = 512
P = T * K + E * B
NB = P // B
FT = 256
NF = F // FT
TB = 1024

NC, NS, L = 2, 16, 16
NW = NC * NS
PP = P + 256
CH = PP // NS
TW = T // NS
GR = P // NW
GC = 48
TCW = T // NW
CC = 32

_mesh = plsc.VectorSubcoreMesh(core_axis_name="c", subcore_axis_name="s")


def _router_body(x_ref, gw_ref, ct_ref):
    xb = x_ref[...]
    gw = gw_ref[...]
    lg = jax.lax.dot_general(gw, xb, (((1,), (1,)), ((), ())),
                             preferred_element_type=jnp.float32)
    iota = jax.lax.broadcasted_iota(jnp.int32, (E, TB), 0)
    m1 = jnp.max(lg, axis=0, keepdims=True)
    i1 = jnp.min(jnp.where(lg == m1, iota, E), axis=0, keepdims=True)
    is1 = iota == i1
    masked = jnp.where(is1, -jnp.inf, lg)
    m2 = jnp.max(masked, axis=0, keepdims=True)
    i2 = jnp.min(jnp.where(masked == m2, iota, E), axis=0, keepdims=True)
    r = jnp.exp(m2 - m1)
    wa = 1.0 / (1.0 + r)
    wb = r * wa
    ct = jnp.where(is1, wa, 0.0) + jnp.where(iota == i2, wb, 0.0)
    ct_ref[...] = ct


def _run_router(x, gate_w):
    return pl.pallas_call(
        _router_body,
        grid=(T // TB,),
        in_specs=[
            pl.BlockSpec((TB, H), lambda i: (i, 0)),
            pl.BlockSpec((E, H), lambda i: (0, 0)),
        ],
        out_specs=pl.BlockSpec((E, TB), lambda i: (0, i)),
        out_shape=jax.ShapeDtypeStruct((E, T), jnp.float32),
    )(x, gate_w)


def _routing_body(ct_hbm, tok_hbm, cof_hbm, pos_hbm, enc_hbm,
                  ctbuf, histbuf, allhist, encbuf,
                  posbuf, idxbuf, tokbuf, cofbuf, zib, zfb,
                  hist_sh, tok_sh, cof_sh):
    cid = lax.axis_index("c")
    sid = lax.axis_index("s")
    on0 = cid == 0

    @pl.when(on0)
    def _():
        t0 = sid * TW
        for e in range(E):
            pltpu.sync_copy(ct_hbm.at[e, pl.ds(t0, TW)],
                            ctbuf.at[pl.ds(e * TW, TW)])
        iota = lax.iota(jnp.int32, L)
        zi = jnp.zeros((L,), jnp.int32)
        zf = jnp.zeros((L,), jnp.float32)

        histv = zi
        for e in range(E):
            acc = zi
            for g in range(TW // L):
                v = ctbuf[pl.ds(e * TW + g * L, L)]
                acc += jnp.where(v > 0.0, 1, 0)
            histv += jnp.where(iota == e, jnp.sum(acc), 0)
        histbuf[pl.ds(0, L)] = histv
        pltpu.sync_copy(histbuf, hist_sh.at[pl.ds(sid * L, L)])
        for i in range(CH // L):
            zib[pl.ds(L * i, L)] = zi
            zfb[pl.ds(L * i, L)] = zf
        pltpu.sync_copy(zib, tok_sh.at[pl.ds(CH * sid, CH)])
        pltpu.sync_copy(zfb, cof_sh.at[pl.ds(CH * sid, CH)])
        for i in range((TW * K) // L):
            posbuf[pl.ds(L * i, L)] = zi + (P - 1)
        for i in range(K * TW // 128):
            for j in range(128 // L):
                idxbuf[i, pl.ds(L * j, L)] = zi + P

    plsc.subcore_barrier()

    @pl.when(on0)
    def _():
        t0 = sid * TW
        iota = lax.iota(jnp.int32, L)
        zi = jnp.zeros((L,), jnp.int32)

        pltpu.sync_copy(hist_sh, allhist)
        total = zi
        before = zi
        widv = zi + sid
        for w in range(NS):
            hv = allhist[pl.ds(w * L, L)]
            total += hv
            before += jnp.where(widv > w, hv, 0)
        pc = ((total + (B - 1)) // B) * B
        offend = plsc.cumsum(pc)
        base = (offend - pc) + before

        @pl.when(sid == 0)
        def _():
            for vb in range(2):
                bstart = (iota + L * vb) * B
                cnt = zi
                for e in range(E):
                    oe = offend.at[jnp.full((L,), e, jnp.int32)].get(
                        mode="promise_in_bounds")
                    cnt += jnp.where(bstart >= oe, 1, 0)
                encv = jnp.where(cnt < E, jnp.minimum(cnt, E - 1), -1)
                encbuf[pl.ds(L * vb, L)] = encv
            pltpu.sync_copy(encbuf, enc_hbm)

        for g in range(TW // L):
            ltok = iota + g * L
            slot = zi
            for e in range(E):
                v = ctbuf[pl.ds(e * TW + g * L, L)]
                sel = v > 0.0
                si = jnp.where(sel, 1, 0)
                basebr = base.at[jnp.full((L,), e, jnp.int32)].get(
                    mode="promise_in_bounds")
                posv = basebr + plsc.cumsum(si) - 1
                sidx = ltok * K + slot
                plsc.store_scatter(posbuf, [sidx], posv, mask=sel)
                plsc.store_scatter(idxbuf, [sidx // 128, sidx % 128],
                                   posv, mask=sel)
                plsc.store_scatter(tokbuf, [sidx // 128, sidx % 128],
                                   ltok + t0, mask=sel)
                plsc.store_scatter(cofbuf, [sidx // 128, sidx % 128],
                                   v, mask=sel)
                slot += si
                base += jnp.where(iota == e, jnp.sum(si), 0)
        for i in range(K * TW // 128):
            pltpu.sync_copy(tokbuf.at[i], tok_sh.at[idxbuf.at[i]])
            pltpu.sync_copy(cofbuf.at[i], cof_sh.at[idxbuf.at[i]])
        pltpu.sync_copy(posbuf, pos_hbm.at[pl.ds(sid * TW * K, TW * K)])

    plsc.subcore_barrier()

    @pl.when(on0)
    def _():
        pltpu.sync_copy(tok_sh.at[pl.ds(CH * sid, CH)], zib)
        pltpu.sync_copy(zib, tok_hbm.at[pl.ds(CH * sid, CH)])
        pltpu.sync_copy(cof_sh.at[pl.ds(CH * sid, CH)], zfb)
        pltpu.sync_copy(zfb, cof_hbm.at[pl.ds(CH * sid, CH)])


def _run_routing(ct):
    kfn = pl.kernel(
        _routing_body,
        out_type=(
            jax.ShapeDtypeStruct((PP,), jnp.int32),
            jax.ShapeDtypeStruct((PP,), jnp.float32),
            jax.ShapeDtypeStruct((T * K,), jnp.int32),
            jax.ShapeDtypeStruct((32,), jnp.int32),
        ),
        mesh=_mesh,
        scratch_types=[
            pltpu.VMEM((E * TW,), jnp.float32),
            pltpu.VMEM((L,), jnp.int32),
            pltpu.VMEM((NS * L,), jnp.int32),
            pltpu.VMEM((2 * L,), jnp.int32),
            pltpu.VMEM((TW * K,), jnp.int32),
            pltpu.VMEM((K * TW // 128, 128), jnp.int32),
            pltpu.VMEM((K * TW // 128, 128), jnp.int32),
            pltpu.VMEM((K * TW // 128, 128), jnp.float32),
            pltpu.VMEM((CH,), jnp.int32),
            pltpu.VMEM((CH,), jnp.float32),
            pltpu.VMEM_SHARED((NS * L,), jnp.int32),
            pltpu.VMEM_SHARED((PP,), jnp.int32),
            pltpu.VMEM_SHARED((PP,), jnp.float32),
        ],
        compiler_params=pltpu.CompilerParams(needs_layout_passes=False),
    )
    return kfn(ct)


def _route_jnp(ct):
    c = ct.T
    mask = c > 0.0
    mi = mask.astype(jnp.int32)
    hist = jnp.sum(mi, axis=0)
    pc = ((hist + B - 1) // B) * B
    offend = jnp.cumsum(pc)
    off = offend - pc
    rank = jnp.cumsum(mi, axis=0) - mi
    posmat = off[None, :] + rank
    scat = jnp.where(mask, posmat, P)
    tvec = jax.lax.broadcasted_iota(jnp.int32, (T, E), 0)
    tok = jnp.zeros((P,), jnp.int32).at[scat.reshape(-1)].set(
        tvec.reshape(-1), mode="drop")
    coeff = jnp.zeros((P,), jnp.float32).at[scat.reshape(-1)].set(
        c.reshape(-1), mode="drop")
    kidx = jnp.cumsum(mi, axis=1) - mi
    prows = jnp.where(mask, tvec * K + kidx, T * K)
    pos = jnp.full((T * K,), P - 1, jnp.int32).at[prows.reshape(-1)].set(
        posmat.reshape(-1).astype(jnp.int32), mode="drop")
    bstart = jnp.arange(NB, dtype=jnp.int32) * B
    be = jnp.sum((bstart[:, None] >= offend[None, :]).astype(jnp.int32), axis=1)
    enc = jnp.where(be < E, jnp.minimum(be, E - 1), -1)
    return tok, coeff, pos, enc


def _gather_body(x_hbm, tok_hbm, xs_hbm, tokw, buf0, buf1, sem0, sem1):
    cid = lax.axis_index("c")
    sid = lax.axis_index("s")
    wid = sid * NC + cid
    r0 = wid * GR
    pltpu.sync_copy(tok_hbm.at[pl.ds(r0, GR)], tokw)
    nchunk = GR // GC
    cps = []
    for j in range(nchunk):
        buf = buf0 if j % 2 == 0 else buf1
        sem = sem0 if j % 2 == 0 else sem1
        cp = pltpu.async_copy(x_hbm.at[tokw.at[pl.ds(j * GC, GC)]], buf, sem)
        cps.append(cp)
        if j >= 1:
            cps[j - 1].wait()
            pb = buf1 if j % 2 == 0 else buf0
            pltpu.sync_copy(pb, xs_hbm.at[pl.ds(r0 + (j - 1) * GC, GC)])
    cps[nchunk - 1].wait()
    lb = buf0 if (nchunk - 1) % 2 == 0 else buf1
    pltpu.sync_copy(lb, xs_hbm.at[pl.ds(r0 + (nchunk - 1) * GC, GC)])


def _run_gather(x, tok):
    kfn = pl.kernel(
        _gather_body,
        out_type=jax.ShapeDtypeStruct((P, H), jnp.float32),
        mesh=_mesh,
        scratch_types=[
            pltpu.VMEM((GR,), jnp.int32),
            pltpu.VMEM((GC, H), jnp.float32),
            pltpu.VMEM((GC, H), jnp.float32),
            pltpu.SemaphoreType.DMA,
            pltpu.SemaphoreType.DMA,
        ],
    )
    return kfn(x, tok)


def _ffn_body(enc_ref, xs_ref, w1_ref, w3_ref, w2_ref, cf_ref, ye_ref):
    f = pl.program_id(1)
    b = pl.program_id(0)
    active = enc_ref[b] >= 0

    @pl.when(f == 0)
    def _():
        ye_ref[...] = jnp.zeros_like(ye_ref)

    @pl.when(active)
    def _():
        xs = xs_ref[...].astype(jnp.bfloat16)
        w1b = w1_ref[0].astype(jnp.bfloat16)
        w3b = w3_ref[0].astype(jnp.bfloat16)
        w2b = w2_ref[0].astype(jnp.bfloat16)
        h1 = jax.lax.dot_general(xs, w1b, (((1,), (1,)), ((), ())),
                                 preferred_element_type=jnp.float32)
        h3 = jax.lax.dot_general(xs, w3b, (((1,), (1,)), ((), ())),
                                 preferred_element_type=jnp.float32)
        g = ((h1 * jax.lax.logistic(h1)) * h3).astype(jnp.bfloat16)
        contrib = jax.lax.dot_general(g, w2b, (((1,), (1,)), ((), ())),
                                      preferred_element_type=jnp.float32)
        ye_ref[...] += contrib

    @pl.when(f == NF - 1)
    def _():
        ye_ref[...] *= cf_ref[...]


def _run_ffn(enc, xs, w1, w3, w2, coeffcol):
    grid_spec = pltpu.PrefetchScalarGridSpec(
        num_scalar_prefetch=1,
        grid=(NB, NF),
        in_specs=[
            pl.BlockSpec((B, H), lambda b, f, s: (b, 0)),
            pl.BlockSpec((1, FT, H),
                         lambda b, f, s: (jnp.maximum(s[b], 0),
                                          jnp.where(s[b] >= 0, f, 0), 0)),
            pl.BlockSpec((1, FT, H),
                         lambda b, f, s: (jnp.maximum(s[b], 0),
                                          jnp.where(s[b] >= 0, f, 0), 0)),
            pl.BlockSpec((1, H, FT),
                         lambda b, f, s: (jnp.maximum(s[b], 0), 0,
                                          jnp.where(s[b] >= 0, f, 0))),
            pl.BlockSpec((B, 1), lambda b, f, s: (b, 0)),
        ],
        out_specs=pl.BlockSpec((B, H), lambda b, f, s: (b, 0)),
    )
    return pl.pallas_call(
        _ffn_body,
        grid_spec=grid_spec,
        out_shape=jax.ShapeDtypeStruct((P, H), jnp.float32),
        compiler_params=pltpu.CompilerParams(
            dimension_semantics=("arbitrary", "arbitrary")),
    )(enc, xs, w1, w3, w2, coeffcol)


def _combine_body(ye_hbm, pos_hbm, out_hbm,
                  posw, ideven, idodd, bufa, bufb, sema, semb):
    cid = lax.axis_index("c")
    sid = lax.axis_index("s")
    wid = sid * NC + cid
    t0 = wid * TCW
    iota = lax.iota(jnp.int32, L)
    pltpu.sync_copy(pos_hbm.at[pl.ds(t0 * K, TCW * K)], posw)
    for g in range(TCW // L):
        base = 2 * L * g
        ideven[pl.ds(L * g, L)] = plsc.load_gather(posw, [base + 2 * iota])
        idodd[pl.ds(L * g, L)] = plsc.load_gather(posw, [base + 2 * iota + 1])
    for j in range(TCW // CC):
        cpa = pltpu.async_copy(
            ye_hbm.at[ideven.at[pl.ds(j * CC, CC)]], bufa, sema)
        cpb = pltpu.async_copy(
            ye_hbm.at[idodd.at[pl.ds(j * CC, CC)]], bufb, semb)
        cpa.wait()
        cpb.wait()

        def _add_row(r, carry):
            for c in range(H // L):
                bufa[r, pl.ds(L * c, L)] = (bufa[r, pl.ds(L * c, L)]
                                            + bufb[r, pl.ds(L * c, L)])
            return carry

        lax.fori_loop(0, CC, _add_row, 0)
        pltpu.sync_copy(bufa, out_hbm.at[pl.ds(t0 + j * CC, CC)])


def _run_combine(ye, pos):
    kfn = pl.kernel(
        _combine_body,
        out_type=jax.ShapeDtypeStruct((T, H), jnp.float32),
        mesh=_mesh,
        scratch_types=[
            pltpu.VMEM((TCW * K,), jnp.int32),
            pltpu.VMEM((TCW,), jnp.int32),
            pltpu.VMEM((TCW,), jnp.int32),
            pltpu.VMEM((CC, H), jnp.float32),
            pltpu.VMEM((CC, H), jnp.float32),
            pltpu.SemaphoreType.DMA,
            pltpu.SemaphoreType.DMA,
        ],
        compiler_params=pltpu.CompilerParams(needs_layout_passes=False),
    )
    return kfn(ye, pos)


@jax.jit
def kernel(x, gate_w, w1, w2, w3):
    ct = _run_router(x, gate_w)
    tok, cof, pos, enc = _run_routing(ct)
    xs = _run_gather(x, tok)
    ye = _run_ffn(enc, xs, w1, w3, w2, cof[:P, None])
    out = _run_combine(ye, pos)
    return out
